# Initial kernel scaffold; baseline (speedup 1.0000x reference)
import jax
import jax.numpy as jnp
from jax.experimental import pallas as pl


def kernel(lattent_codes, object_labels, means, components):
    raise NotImplementedError("write your pallas kernel here")


# speedup vs baseline: 12.2948x; 12.2948x over previous
import jax
import jax.numpy as jnp
from jax.experimental import pallas as pl
from jax.experimental.pallas import tpu as pltpu

B = 1024
P = 32
N = 20
NP = N * P
D = 6144
N_PAD = 24
DBLK = 1024


def _decode_kernel(lab_ref, lat_ref, comp_ref, means_ref, out_ref, e_ref, h_ref):
    @pl.when(pl.program_id(0) == 0)
    def _build():
        lab = lab_ref[:, :1]
        cls = jax.lax.broadcasted_iota(jnp.int32, (B, NP), 1) // P
        lat_t = jnp.concatenate([lat_ref[...]] * N, axis=1)
        e_ref[...] = jnp.where(cls == lab, lat_t, 0.0)
        hcls = jax.lax.broadcasted_iota(jnp.int32, (B, N_PAD), 1)
        h_ref[...] = jnp.where(hcls == lab, 1.0, 0.0)

    out_ref[...] = (
        jnp.dot(e_ref[...], comp_ref[...], preferred_element_type=jnp.float32)
        + jnp.dot(h_ref[...], means_ref[...], preferred_element_type=jnp.float32)
    )


def kernel(lattent_codes, object_labels, means, components):
    comp2d = components.reshape(NP, D)
    labels_b = jnp.broadcast_to(
        object_labels.astype(jnp.int32)[:, None], (B, 128)
    )
    means_pad = jnp.pad(means, ((0, N_PAD - N), (0, 0)))

    return pl.pallas_call(
        _decode_kernel,
        grid=(D // DBLK,),
        in_specs=[
            pl.BlockSpec((B, 128), lambda i: (0, 0)),
            pl.BlockSpec((B, P), lambda i: (0, 0)),
            pl.BlockSpec((NP, DBLK), lambda i: (0, i)),
            pl.BlockSpec((N_PAD, DBLK), lambda i: (0, i)),
        ],
        out_specs=pl.BlockSpec((B, DBLK), lambda i: (0, i)),
        out_shape=jax.ShapeDtypeStruct((B, D), jnp.float32),
        scratch_shapes=[
            pltpu.VMEM((B, NP), jnp.float32),
            pltpu.VMEM((B, N_PAD), jnp.float32),
        ],
    )(labels_b, lattent_codes, comp2d, means_pad)


# speedup vs baseline: 12.4137x; 1.0097x over previous
import jax
import jax.numpy as jnp
from jax.experimental import pallas as pl
from jax.experimental.pallas import tpu as pltpu

B = 1024
P = 32
N = 20
NP = N * P
D = 6144
N_PAD = 24
DBLK = 1024


def _decode_kernel(lab_ref, lat_ref, comp_ref, means_ref, out_ref, e_ref, h_ref):
    @pl.when(pl.program_id(0) == 0)
    def _build():
        lab = lab_ref[:, :1]
        cls = jax.lax.broadcasted_iota(jnp.int32, (B, NP), 1) // P
        lat_t = jnp.concatenate([lat_ref[...]] * N, axis=1)
        e_ref[...] = jnp.where(cls == lab, lat_t, 0.0).astype(jnp.bfloat16)
        hcls = jax.lax.broadcasted_iota(jnp.int32, (B, N_PAD), 1)
        h_ref[...] = jnp.where(hcls == lab, 1.0, 0.0)

    out_ref[...] = (
        jnp.dot(
            e_ref[...],
            comp_ref[...].astype(jnp.bfloat16),
            preferred_element_type=jnp.float32,
        )
        + jnp.dot(h_ref[...], means_ref[...], preferred_element_type=jnp.float32)
    )


def kernel(lattent_codes, object_labels, means, components):
    comp2d = components.reshape(NP, D)
    labels_b = jnp.broadcast_to(
        object_labels.astype(jnp.int32)[:, None], (B, 128)
    )
    means_pad = jnp.pad(means, ((0, N_PAD - N), (0, 0)))

    return pl.pallas_call(
        _decode_kernel,
        grid=(D // DBLK,),
        in_specs=[
            pl.BlockSpec((B, 128), lambda i: (0, 0)),
            pl.BlockSpec((B, P), lambda i: (0, 0)),
            pl.BlockSpec((NP, DBLK), lambda i: (0, i)),
            pl.BlockSpec((N_PAD, DBLK), lambda i: (0, i)),
        ],
        out_specs=pl.BlockSpec((B, DBLK), lambda i: (0, i)),
        out_shape=jax.ShapeDtypeStruct((B, D), jnp.float32),
        scratch_shapes=[
            pltpu.VMEM((B, NP), jnp.bfloat16),
            pltpu.VMEM((B, N_PAD), jnp.float32),
        ],
    )(labels_b, lattent_codes, comp2d, means_pad)
